# baseline (speedup 1.0000x reference)
import jax
import jax.numpy as jnp
from jax import lax
from jax.experimental import pallas as pl
from jax.experimental.pallas import tpu as pltpu

MASK_LEN = 4915
B, T, D = 4, 8192, 1024

CP = 416
NCH = 8
SUF0 = T - CP * NCH
FT = 1024
FILL_FULL = 4
REM_BITS = (512, 256, 128, 64, 32, 16, 8)


def _body(start_ref, x_hbm, mt_ref, out_hbm, mask_ref, tile, ebuf, sem_c, sem_f, sem_e):
    L = MASK_LEN

    copies = []
    for b in range(B):
        s = start_ref[b]
        e0 = pl.multiple_of(s & -8, 8)
        e1 = pl.multiple_of((s + L) & -8, 8)
        for jc in range(NCH):
            off = jc * CP
            d = pltpu.make_async_copy(x_hbm.at[pl.ds(b, 1), pl.ds(off, CP)],
                                      out_hbm.at[pl.ds(b, 1), pl.ds(off, CP)],
                                      sem_c)
            copies.append((off < s, d))
        for jc in range(NCH):
            off = SUF0 + jc * CP
            d = pltpu.make_async_copy(x_hbm.at[pl.ds(b, 1), pl.ds(off, CP)],
                                      out_hbm.at[pl.ds(b, 1), pl.ds(off, CP)],
                                      sem_c)
            copies.append((off + CP > s + L, d))
        for e, eo in ((0, e0), (1, e1)):
            k = 2 * b + e
            pltpu.make_async_copy(x_hbm.at[pl.ds(b, 1), pl.ds(eo, 8)],
                                  ebuf.at[pl.ds(k, 1)], sem_e.at[k]).start()

    for cond, d in copies:
        pl.when(cond)(d.start)

    tile[...] = jnp.broadcast_to(mt_ref[...][None], (1, FT, D))
    pos_row = lax.broadcasted_iota(jnp.int32, (1, T), 1)
    pos_col = lax.broadcasted_iota(jnp.int32, (8, 1), 0)
    edge_outs = []
    for b in range(B):
        s = start_ref[b]
        mask_ref[b : b + 1, :] = (pos_row >= s) & (pos_row < s + L)
        e0 = pl.multiple_of(s & -8, 8)
        e1 = pl.multiple_of((s + L) & -8, 8)
        for e, eo in ((0, e0), (1, e1)):
            k = 2 * b + e
            pltpu.make_async_copy(x_hbm.at[pl.ds(b, 1), pl.ds(eo, 8)],
                                  ebuf.at[pl.ds(k, 1)], sem_e.at[k]).wait()
            pos = eo + pos_col
            m = (pos >= s) & (pos < s + L)
            ebuf[k] = jnp.where(m, mt_ref[...], ebuf[k])
            edge_outs.append(
                pltpu.make_async_copy(ebuf.at[pl.ds(k, 1)],
                                      out_hbm.at[pl.ds(b, 1), pl.ds(eo, 8)],
                                      sem_e.at[k]))

    for cond, d in copies:
        pl.when(cond)(d.wait)

    fills = []
    for b in range(B):
        s = start_ref[b]
        base = pl.multiple_of((s & -8) + 8, 8)
        e1 = pl.multiple_of((s + L) & -8, 8)
        rem = (e1 - base) - FILL_FULL * FT
        for i in range(FILL_FULL):
            d = pltpu.make_async_copy(
                tile.at[pl.ds(0, 1), pl.ds(0, FT)],
                out_hbm.at[pl.ds(b, 1), pl.ds(pl.multiple_of(base + i * FT, 8), FT)],
                sem_f)
            fills.append((None, d))
        for sg in REM_BITS:
            off = pl.multiple_of(base + FILL_FULL * FT + (rem & ~(2 * sg - 1)), 8)
            d = pltpu.make_async_copy(tile.at[pl.ds(0, 1), pl.ds(0, sg)],
                                      out_hbm.at[pl.ds(b, 1), pl.ds(off, sg)],
                                      sem_f)
            fills.append(((rem & sg) != 0, d))

    for cond, d in fills:
        if cond is None:
            d.start()
        else:
            pl.when(cond)(d.start)
    for d in edge_outs:
        d.start()

    for cond, d in fills:
        if cond is None:
            d.wait()
        else:
            pl.when(cond)(d.wait)
    for d in edge_outs:
        d.wait()


def kernel(x, start_idx, mask_token):
    start_idx = start_idx.astype(jnp.int32)
    x_masked, mask = pl.pallas_call(
        _body,
        in_specs=[
            pl.BlockSpec(memory_space=pltpu.MemorySpace.SMEM),
            pl.BlockSpec(memory_space=pl.ANY),
            pl.BlockSpec(memory_space=pltpu.MemorySpace.VMEM),
        ],
        out_specs=[
            pl.BlockSpec(memory_space=pl.ANY),
            pl.BlockSpec(memory_space=pltpu.MemorySpace.VMEM),
        ],
        out_shape=[
            jax.ShapeDtypeStruct((B, T, D), jnp.float32),
            jax.ShapeDtypeStruct((B, T), jnp.bool_),
        ],
        scratch_shapes=[
            pltpu.VMEM((1, FT, D), jnp.float32),
            pltpu.VMEM((2 * B, 8, D), jnp.float32),
            pltpu.SemaphoreType.DMA,
            pltpu.SemaphoreType.DMA,
            pltpu.SemaphoreType.DMA((2 * B,)),
        ],
    )(start_idx, x, mask_token.reshape(1, D))
    return (x_masked, mask)


# speedup vs baseline: 23.5737x; 23.5737x over previous
import jax
import jax.numpy as jnp
from jax import lax
from jax.experimental import pallas as pl
from jax.experimental.pallas import tpu as pltpu

MASK_LEN = 4915
B, T, D = 4, 8192, 1024
BT = 512


def _x_index(b, t, start_ref):
    s = start_ref[b]
    sb = s // BT
    eb = (s + MASK_LEN - 1) // BT
    interior = (t > sb) & (t < eb)
    return (b, jnp.where(interior, sb, t), 0)


def _body(start_ref, x_ref, mt_ref, out_ref):
    b = pl.program_id(0)
    t = pl.program_id(1)
    s = start_ref[b]
    base = t * BT
    sb = s // BT
    eb = (s + MASK_LEN - 1) // BT
    interior = (t > sb) & (t < eb)

    @pl.when(interior)
    def _():
        out_ref[0] = jnp.broadcast_to(mt_ref[...], (BT, D))

    @pl.when(jnp.logical_not(interior))
    def _():
        pos = lax.broadcasted_iota(jnp.int32, (BT, 1), 0) + base
        m = (pos >= s) & (pos < s + MASK_LEN)
        out_ref[0] = jnp.where(m, mt_ref[...], x_ref[0])


def _mask_body(start_ref, mask_ref):
    pos = lax.broadcasted_iota(jnp.int32, (1, T), 1)
    for b in range(B):
        s = start_ref[b]
        mask_ref[b : b + 1, :] = (pos >= s) & (pos < s + MASK_LEN)


def kernel(x, start_idx, mask_token):
    start_idx = start_idx.astype(jnp.int32)
    grid_spec = pltpu.PrefetchScalarGridSpec(
        num_scalar_prefetch=1,
        grid=(B, T // BT),
        in_specs=[
            pl.BlockSpec((1, BT, D), _x_index),
            pl.BlockSpec((1, D), lambda b, t, s: (0, 0)),
        ],
        out_specs=[
            pl.BlockSpec((1, BT, D), lambda b, t, s: (b, t, 0)),
        ],
    )
    x_masked = pl.pallas_call(
        _body,
        grid_spec=grid_spec,
        out_shape=[jax.ShapeDtypeStruct((B, T, D), jnp.float32)],
    )(start_idx, x, mask_token.reshape(1, D))[0]
    mask = pl.pallas_call(
        _mask_body,
        in_specs=[pl.BlockSpec(memory_space=pltpu.MemorySpace.SMEM)],
        out_shape=jax.ShapeDtypeStruct((B, T), jnp.bool_),
    )(start_idx)
    return (x_masked, mask)
